# baseline (speedup 1.0000x reference)
import functools

import jax
import jax.numpy as jnp
from jax import lax
from jax.experimental import pallas as pl
from jax.experimental.pallas import tpu as pltpu
from jax.experimental.pallas import tpu_sc as plsc

J, M, S, D = 1024, 128, 128, 128
_EPS = 1e-8

_NC, _NS = 2, 16
_NW = _NC * _NS
_JOB_PER_W = J // _NW
_NH = _NW // 2
_SM_PER_W = M // _NH


def _gather_body(jidx_hbm, midx_hbm, sidx_hbm, jtab_hbm, mtab_hbm, stab_hbm,
                 jout_hbm, mout_hbm, sout_hbm,
                 jidx_v, jrows_v, midx_v, mrows_v, sidx_v, srows_v,
                 sem_a, sem_b, sem_c):
    wid = lax.axis_index("s") * _NC + lax.axis_index("c")
    hid = lax.rem(wid, _NH)

    jb = pl.multiple_of(wid * _JOB_PER_W, 8)
    sb = pl.multiple_of(hid * _SM_PER_W, 8)

    c1 = pltpu.async_copy(jidx_hbm.at[pl.ds(jb, _JOB_PER_W)], jidx_v, sem_a)
    c2 = pltpu.async_copy(midx_hbm.at[pl.ds(sb, _SM_PER_W)], midx_v, sem_b)
    c3 = pltpu.async_copy(sidx_hbm.at[pl.ds(sb, _SM_PER_W)], sidx_v, sem_c)
    c2.wait()
    g2 = pltpu.async_copy(mtab_hbm.at[midx_v], mrows_v, sem_b)
    c3.wait()
    g3 = pltpu.async_copy(stab_hbm.at[sidx_v], srows_v, sem_c)
    c1.wait()
    g1 = pltpu.async_copy(jtab_hbm.at[jidx_v], jrows_v, sem_a)
    g2.wait()
    w2 = pltpu.async_copy(mrows_v, mout_hbm.at[pl.ds(sb, _SM_PER_W)], sem_b)
    g3.wait()
    w3 = pltpu.async_copy(srows_v, sout_hbm.at[pl.ds(sb, _SM_PER_W)], sem_c)
    g1.wait()
    w1 = pltpu.async_copy(jrows_v, jout_hbm.at[pl.ds(jb, _JOB_PER_W)], sem_a)
    w2.wait()
    w3.wait()
    w1.wait()


@functools.cache
def _gather_sc():
    return pl.kernel(
        _gather_body,
        mesh=plsc.VectorSubcoreMesh(core_axis_name="c", subcore_axis_name="s"),
        out_type=[
            jax.ShapeDtypeStruct((J, D), jnp.float32),
            jax.ShapeDtypeStruct((M, D), jnp.float32),
            jax.ShapeDtypeStruct((S, D), jnp.float32),
        ],
        scratch_types=[
            pltpu.VMEM((_JOB_PER_W,), jnp.int32),
            pltpu.VMEM((_JOB_PER_W, D), jnp.float32),
            pltpu.VMEM((_SM_PER_W,), jnp.int32),
            pltpu.VMEM((_SM_PER_W, D), jnp.float32),
            pltpu.VMEM((_SM_PER_W,), jnp.int32),
            pltpu.VMEM((_SM_PER_W, D), jnp.float32),
            pltpu.SemaphoreType.DMA,
            pltpu.SemaphoreType.DMA,
            pltpu.SemaphoreType.DMA,
        ],
    )


def _sim_body(jemb_ref, memb_ref, semb_ref, out_ref):
    je = jemb_ref[...]
    me = memb_ref[...]
    se = semb_ref[...]
    jn = jnp.sqrt(jnp.sum(je * je, axis=1))
    mn = jnp.sqrt(jnp.sum(me * me, axis=1))
    sn = jnp.sqrt(jnp.sum(se * se, axis=1))
    jm_dot = lax.dot_general(je, me, (((1,), (1,)), ((), ())),
                             preferred_element_type=jnp.float32)
    jm = jm_dot / jnp.maximum(jn[:, None] * mn[None, :], _EPS)
    ms_dot = lax.dot_general(me, se, (((1,), (1,)), ((), ())),
                             preferred_element_type=jnp.float32)
    ms = ms_dot / jnp.maximum(mn[:, None] * sn[None, :], _EPS)
    out_ref[...] = jm[:, :, None] * ms[None, :, :]


_JB = 128


def kernel(job_indices, major_indices, subject_indices,
           job_table, major_table, subject_table):
    jemb, memb, semb = _gather_sc()(
        job_indices.astype(jnp.int32), major_indices.astype(jnp.int32),
        subject_indices.astype(jnp.int32),
        job_table, major_table, subject_table)
    out = pl.pallas_call(
        _sim_body,
        grid=(J // _JB,),
        in_specs=[
            pl.BlockSpec((_JB, D), lambda i: (i, 0)),
            pl.BlockSpec((M, D), lambda i: (0, 0)),
            pl.BlockSpec((S, D), lambda i: (0, 0)),
        ],
        out_specs=pl.BlockSpec((_JB, M, S), lambda i: (i, 0, 0)),
        out_shape=jax.ShapeDtypeStruct((J, M, S), jnp.float32),
    )(jemb, memb, semb)
    return out.reshape(-1)
